# baseline (speedup 1.0000x reference)
import functools

import jax
import jax.numpy as jnp
from jax import lax
from jax.experimental import pallas as pl
from jax.experimental.pallas import tpu as pltpu
from jax.experimental.pallas import tpu_sc as plsc

NUM_MODELS = 2
NUM_BOXES = 1000000
DIM = 32
BATCH = 4096

_INFO = plsc.get_sparse_core_info()
_NC = _INFO.num_cores
_NS = _INFO.num_subcores
_NW = _NC * _NS
_L = _INFO.num_lanes

_P = NUM_MODELS * DIM
_BS = 2560
_BT = BATCH - _BS
_TCB = 8
_BPW = _BS // _NW
_NSLOT = 6
_MAIN = (_BPW // _NSLOT) * _NSLOT

_GROUP_ROW0 = (0, 16, 64, 80)


def _sc_body(z_hbm, ld_hbm, idx_hbm, out_hbm, idx_v, zblk, lblk, obuf, *sems):
    semz = sems[:_NSLOT]
    seml = sems[_NSLOT:]
    wid = lax.axis_index("c") * _NS + lax.axis_index("s")
    b0 = wid * _BPW

    pltpu.sync_copy(idx_hbm.at[pl.ds(b0, _BPW)], idx_v.at[pl.ds(0, _BPW)])

    iota = lax.iota(jnp.int32, _L)

    def enqueue(box, slot):
        chunk = idx_v[pl.ds(box, _L)]
        c0 = pl.multiple_of((chunk[0] >> 7) << 7, 128)
        pltpu.async_copy(z_hbm.at[:, pl.ds(c0, 128)], zblk.at[slot], semz[slot])
        pltpu.async_copy(ld_hbm.at[:, pl.ds(c0, 128)], lblk.at[slot], seml[slot])

    def drain(slot):
        pltpu.make_async_copy(z_hbm.at[:, pl.ds(0, 128)],
                              zblk.at[slot], semz[slot]).wait()
        pltpu.make_async_copy(ld_hbm.at[:, pl.ds(0, 128)],
                              lblk.at[slot], seml[slot]).wait()

    def compute(box, slot):
        chunk = idx_v[pl.ds(box, _L)]
        col = jnp.full((_L,), chunk[0] & 127, jnp.int32)
        kk = jnp.full((_L,), slot, jnp.int32)
        jj = jnp.full((_L,), box, jnp.int32)
        for g in range(_P // _L):
            rows = iota + g * _L
            zv = plsc.load_gather(zblk, [kk, rows, col])
            lv = plsc.load_gather(lblk, [kk, rows, col])
            ev = zv + jnp.exp(lv)
            orow = iota + _GROUP_ROW0[g]
            plsc.store_scatter(obuf, [orow, jj], zv)
            plsc.store_scatter(obuf, [orow + DIM, jj], ev)

    for s in range(_NSLOT - 1):
        enqueue(s, s)

    def step(it, carry):
        for slot in range(_NSLOT):
            box = _NSLOT * it + slot

            @pl.when(box + _NSLOT - 1 < _BPW)
            def _(box=box, slot=slot):
                enqueue(box + _NSLOT - 1, (slot + _NSLOT - 1) % _NSLOT)

            drain(slot)
            compute(box, slot)
        return carry

    lax.fori_loop(0, _MAIN // _NSLOT, step, 0)

    for box in range(_MAIN, _BPW):
        drain(box % _NSLOT)
        compute(box, box % _NSLOT)

    pltpu.sync_copy(obuf, out_hbm.at[wid])


def _tc_body(win_ref, col_ref, *refs):
    z_refs = refs[:_TCB]
    l_refs = refs[_TCB:2 * _TCB]
    out_ref = refs[2 * _TCB]
    i = pl.program_id(0)
    cols = []
    for k in range(_TCB):
        c = col_ref[i * _TCB + k]
        sh = (128 - c) & 127
        zr = pltpu.roll(z_refs[k][...], sh, axis=1)
        lr = pltpu.roll(l_refs[k][...], sh, axis=1)
        zcol = zr[:, 0:1]
        scol = zcol + jnp.exp(lr[:, 0:1])
        cols.append(jnp.concatenate(
            [zcol[:DIM], scol[:DIM], zcol[DIM:], scol[DIM:]], axis=0))
    out_ref[...] = jnp.concatenate(cols, axis=1)[None]


def _win_spec(k):
    return pl.BlockSpec(
        (_P, 128), lambda i, win, col, k=k: (0, win[i * _TCB + k]))


@jax.jit
def kernel(z, logdelta, box_indices):
    zf = z.transpose(0, 2, 1).reshape(_P, NUM_BOXES)
    lf = logdelta.transpose(0, 2, 1).reshape(_P, NUM_BOXES)
    idx = box_indices.astype(jnp.int32)

    mesh = plsc.VectorSubcoreMesh(core_axis_name="c", subcore_axis_name="s")
    sc_out = pl.kernel(
        _sc_body,
        mesh=mesh,
        compiler_params=pltpu.CompilerParams(needs_layout_passes=False),
        out_type=jax.ShapeDtypeStruct((_NW, 2 * _P, _BPW), jnp.float32),
        scratch_types=[
            pltpu.VMEM((_BPW + _L,), jnp.int32),
            pltpu.VMEM((_NSLOT, _P, 128), jnp.float32),
            pltpu.VMEM((_NSLOT, _P, 128), jnp.float32),
            pltpu.VMEM((2 * _P, _BPW), jnp.float32),
        ] + [pltpu.SemaphoreType.DMA] * (2 * _NSLOT),
    )(zf, lf, idx[:_BS])

    idx_tc = idx[_BS:]
    tc_out = pl.pallas_call(
        _tc_body,
        grid_spec=pltpu.PrefetchScalarGridSpec(
            num_scalar_prefetch=2,
            grid=(_BT // _TCB,),
            in_specs=[_win_spec(k) for k in range(_TCB)] * 2,
            out_specs=pl.BlockSpec((1, 2 * _P, _TCB),
                                   lambda i, win, col: (i, 0, 0)),
        ),
        out_shape=jax.ShapeDtypeStruct((_BT // _TCB, 2 * _P, _TCB),
                                       jnp.float32),
    )(idx_tc >> 7, idx_tc & 127, *([zf] * _TCB), *([lf] * _TCB))

    sc2d = sc_out.transpose(1, 0, 2).reshape(2 * _P, _BS)
    tc2d = tc_out.transpose(1, 0, 2).reshape(2 * _P, _BT)
    out = jnp.concatenate([sc2d, tc2d], axis=1)
    return out.reshape(NUM_MODELS, 2, DIM, BATCH).transpose(0, 3, 1, 2)
